# baseline (speedup 1.0000x reference)
import functools

import jax
import jax.numpy as jnp
from jax import lax
from jax.experimental import pallas as pl
from jax.experimental.pallas import tpu as pltpu
from jax.experimental.pallas import tpu_sc as plsc

B = 4096
C = 1000
D = 512
LAMBDA1 = 0.005
LAMBDA2 = 1.0

NC = 2
NS = 16
NW = NC * NS
BPW = B // NW
R = 64
NCH = BPW // R
LN = 16
DV = D // LN
CP = 1024

_mesh = plsc.VectorSubcoreMesh(core_axis_name="c", subcore_axis_name="s")


@functools.partial(
    pl.kernel,
    out_type=[
        jax.ShapeDtypeStruct((NW * LN,), jnp.float32),
        jax.ShapeDtypeStruct((2 * LN,), jnp.float32),
    ],
    mesh=_mesh,
    scratch_types=[
        pltpu.VMEM((NCH, R), jnp.int32),
        pltpu.VMEM((R, D), jnp.float32),
        pltpu.VMEM((R, D), jnp.float32),
        pltpu.VMEM((R, 128), jnp.float32),
        pltpu.VMEM((CP,), jnp.float32),
        pltpu.VMEM((LN,), jnp.float32),
        pltpu.SemaphoreType.DMA,
    ],
)
def _sc_loss(x_hbm, y3_hbm, w_hbm, gb_hbm, gp_hbm, out1_hbm, out2_hbm,
             idx_v, xbuf, wbuf, gbv, gpv, stage, sem):
    wid = lax.axis_index("s") * NC + lax.axis_index("c")
    base = wid * BPW

    pltpu.sync_copy(y3_hbm.at[wid], idx_v)

    acc = jnp.zeros((LN,), jnp.float32)
    for c in range(NCH):
        pltpu.async_copy(x_hbm.at[pl.ds(base + c * R, R)], xbuf, sem).wait()
        pltpu.async_copy(w_hbm.at[idx_v.at[c]], wbuf, sem).wait()
        pltpu.async_copy(gb_hbm.at[idx_v.at[c]], gbv, sem).wait()

        def row_body(r, a):
            g16 = gbv[r, pl.ds(0, LN)]
            for dd in range(DV):
                xv = xbuf[r, pl.ds(dd * LN, LN)]
                wv = wbuf[r, pl.ds(dd * LN, LN)]
                t = xv - g16 * wv
                a = a + t * t
            return a

        acc = lax.fori_loop(0, R, row_body, acc)

    stage[...] = acc
    pltpu.sync_copy(stage, out1_hbm.at[pl.ds(wid * LN, LN)])

    @pl.when(wid == 0)
    def _():
        pltpu.sync_copy(gp_hbm, gpv)

        def g_body(i, carry):
            s, s2 = carry
            gv = gpv[pl.ds(i * LN, LN)]
            return s + gv, s2 + gv * gv

        s, s2 = lax.fori_loop(0, CP // LN, g_body,
                              (jnp.zeros((LN,), jnp.float32),
                               jnp.zeros((LN,), jnp.float32)))
        stage[...] = s
        pltpu.sync_copy(stage, out2_hbm.at[pl.ds(0, LN)])
        stage[...] = s2
        pltpu.sync_copy(stage, out2_hbm.at[pl.ds(LN, LN)])


def kernel(output_features, y_truth, fixed_weights, centers_gamma):
    gflat = centers_gamma.reshape(-1).astype(jnp.float32)
    gpad = jnp.pad(gflat, (0, CP - C))
    gbcast = jnp.broadcast_to(gpad[:, None], (CP, 128))
    y3 = y_truth.reshape(NW, NCH, R)

    out1, out2 = _sc_loss(output_features, y3, fixed_weights, gbcast, gpad)

    loss1 = 0.5 * jnp.sum(out1) / B
    sg = jnp.sum(out2[:LN])
    sg2 = jnp.sum(out2[LN:])
    L = 2.0 * (C - 1) * sg2 + 2.0 * (sg * sg - sg2) / (C - 1)
    loss2 = C * (C - 1) / L
    return LAMBDA1 * loss1 + LAMBDA2 * loss2


# speedup vs baseline: 1.1190x; 1.1190x over previous
import functools

import jax
import jax.numpy as jnp
from jax import lax
from jax.experimental import pallas as pl
from jax.experimental.pallas import tpu as pltpu
from jax.experimental.pallas import tpu_sc as plsc

B = 4096
C = 1000
D = 512
LAMBDA1 = 0.005
LAMBDA2 = 1.0

NC = 2
NS = 16
NW = NC * NS
BPW = B // NW
R = 32
NCH = BPW // R
LN = 16
DV = D // LN
CP = 1024

_mesh = plsc.VectorSubcoreMesh(core_axis_name="c", subcore_axis_name="s")


@functools.partial(
    pl.kernel,
    out_type=[
        jax.ShapeDtypeStruct((NW * LN,), jnp.float32),
        jax.ShapeDtypeStruct((2 * LN,), jnp.float32),
    ],
    mesh=_mesh,
    scratch_types=[
        pltpu.VMEM((NCH, R), jnp.int32),
        pltpu.VMEM((2, R, D), jnp.float32),
        pltpu.VMEM((2, R, D), jnp.float32),
        pltpu.VMEM((2, R, 128), jnp.float32),
        pltpu.VMEM((CP,), jnp.float32),
        pltpu.VMEM((LN,), jnp.float32),
        pltpu.SemaphoreType.DMA((2,)),
        pltpu.SemaphoreType.DMA((2,)),
        pltpu.SemaphoreType.DMA((2,)),
    ],
)
def _sc_loss(x_hbm, y3_hbm, w_hbm, gb_hbm, gp_hbm, out1_hbm, out2_hbm,
             idx_v, xbuf, wbuf, gbv, gpv, stage, semx, semw, semg):
    wid = lax.axis_index("s") * NC + lax.axis_index("c")
    base = wid * BPW

    pltpu.sync_copy(y3_hbm.at[wid], idx_v)

    def start_chunk(c, b):
        return (
            pltpu.async_copy(x_hbm.at[pl.ds(base + c * R, R)], xbuf.at[b],
                             semx.at[b]),
            pltpu.async_copy(w_hbm.at[idx_v.at[c]], wbuf.at[b], semw.at[b]),
            pltpu.async_copy(gb_hbm.at[idx_v.at[c]], gbv.at[b], semg.at[b]),
        )

    inflight = [None, None]
    inflight[0] = start_chunk(0, 0)
    acc = jnp.zeros((LN,), jnp.float32)
    for c in range(NCH):
        b = c & 1
        if c + 1 < NCH:
            inflight[(c + 1) & 1] = start_chunk(c + 1, (c + 1) & 1)
        for cp in inflight[b]:
            cp.wait()
        xb, wb, gv = xbuf.at[b], wbuf.at[b], gbv.at[b]

        def row_body(r, a):
            g16 = gv[r, pl.ds(0, LN)]
            for dd in range(DV):
                xv = xb[r, pl.ds(dd * LN, LN)]
                wv = wb[r, pl.ds(dd * LN, LN)]
                t = xv - g16 * wv
                a = a + t * t
            return a

        acc = lax.fori_loop(0, R, row_body, acc)

    stage[...] = acc
    pltpu.sync_copy(stage, out1_hbm.at[pl.ds(wid * LN, LN)])

    @pl.when(wid == 0)
    def _():
        pltpu.sync_copy(gp_hbm, gpv)

        def g_body(i, carry):
            s, s2 = carry
            gv = gpv[pl.ds(i * LN, LN)]
            return s + gv, s2 + gv * gv

        s, s2 = lax.fori_loop(0, CP // LN, g_body,
                              (jnp.zeros((LN,), jnp.float32),
                               jnp.zeros((LN,), jnp.float32)))
        stage[...] = s
        pltpu.sync_copy(stage, out2_hbm.at[pl.ds(0, LN)])
        stage[...] = s2
        pltpu.sync_copy(stage, out2_hbm.at[pl.ds(LN, LN)])


def kernel(output_features, y_truth, fixed_weights, centers_gamma):
    gflat = centers_gamma.reshape(-1).astype(jnp.float32)
    gpad = jnp.pad(gflat, (0, CP - C))
    gbcast = jnp.broadcast_to(gpad[:, None], (CP, 128))
    y3 = y_truth.reshape(NW, NCH, R)

    out1, out2 = _sc_loss(output_features, y3, fixed_weights, gbcast, gpad)

    loss1 = 0.5 * jnp.sum(out1) / B
    sg = jnp.sum(out2[:LN])
    sg2 = jnp.sum(out2[LN:])
    L = 2.0 * (C - 1) * sg2 + 2.0 * (sg * sg - sg2) / (C - 1)
    loss2 = C * (C - 1) / L
    return LAMBDA1 * loss1 + LAMBDA2 * loss2
